# baseline (speedup 1.0000x reference)
import functools

import jax
import jax.numpy as jnp
from jax import lax
from jax.experimental import pallas as pl
from jax.experimental.pallas import tpu as pltpu
from jax.experimental.pallas import tpu_sc as plsc

N = 100016
E = 3200512
G = 2632

NC = 2
NS = 16
NW = NC * NS

CW = 128
BR = 8
BLK = BR * CW
NBLK = E // BLK
TAIL = E - NBLK * BLK
TAIL_R = TAIL // CW
ATR = E // CW
BASE_BLK = NBLK // NW
EXTRA = NBLK - BASE_BLK * NW

ZS = 6256
NP = NS * ZS

GT = 88
NT2 = GT * 38
NBUF = 96 * 38

P_WE, P_BE, P_ROOT, P_BC = 0, 1, 2, 3
P_W1, P_B1, P_W2, P_B2, P_W3, P_B3 = 4, 156, 160, 176, 180, 184


def _make_stage1():
    mesh = plsc.VectorSubcoreMesh(core_axis_name="c", subcore_axis_name="s")

    @functools.partial(
        pl.kernel,
        out_type=jax.ShapeDtypeStruct((NC * NP,), jnp.float32),
        mesh=mesh,
        compiler_params=pltpu.CompilerParams(needs_layout_passes=False),
        scratch_types=[
            pltpu.VMEM((N,), jnp.float32),
            [pltpu.VMEM((2, BLK), jnp.int32) for _ in range(2)],
            [pltpu.VMEM((BR, CW), jnp.float32) for _ in range(2)],
            [pltpu.VMEM((BLK,), jnp.float32) for _ in range(2)],
            [pltpu.VMEM((BLK,), jnp.int32) for _ in range(2)],
            pltpu.VMEM((ZS,), jnp.float32),
            pltpu.VMEM_SHARED((NP,), jnp.float32),
            [pltpu.SemaphoreType.DMA for _ in range(2)],
            [pltpu.SemaphoreType.DMA for _ in range(2)],
        ],
    )
    def stage1(ei_h, attr_h, x_h, out_h,
               x_v, ei_v, attr_v, msg_v, didx_v, z_v, agg_s,
               in_sem, sc_sem):
        c = lax.axis_index("c")
        s = lax.axis_index("s")
        t = c * NS + s

        nblk = jnp.where(t < EXTRA, BASE_BLK + 1, BASE_BLK)
        blk0 = t * BASE_BLK + jnp.minimum(t, EXTRA)

        def start_in(slot, blk):
            e0 = pl.multiple_of(blk * BLK, BLK)
            r0 = pl.multiple_of(blk * BR, BR)
            pltpu.async_copy(ei_h.at[pl.ds(0, 2), pl.ds(e0, BLK)],
                             ei_v[slot], in_sem[slot])
            pltpu.async_copy(attr_h.at[pl.ds(r0, BR)],
                             attr_v[slot], in_sem[slot])

        def wait_in(slot, blk):
            e0 = pl.multiple_of(blk * BLK, BLK)
            r0 = pl.multiple_of(blk * BR, BR)
            pltpu.make_async_copy(ei_h.at[pl.ds(0, 2), pl.ds(e0, BLK)],
                                  ei_v[slot], in_sem[slot]).wait()
            pltpu.make_async_copy(attr_h.at[pl.ds(r0, BR)],
                                  attr_v[slot], in_sem[slot]).wait()

        start_in(0, blk0)
        start_in(1, blk0 + 1)

        zeros16 = jnp.zeros((16,), jnp.float32)

        def zbody(i, carry):
            z_v[pl.ds(i * 16, 16)] = zeros16
            return carry

        lax.fori_loop(0, ZS // 16, zbody, 0)
        pltpu.sync_copy(z_v, agg_s.at[pl.ds(s * ZS, ZS)])
        plsc.subcore_barrier()

        pltpu.sync_copy(x_h, x_v)

        def compute(slot):
            for r in range(BR):
                for v in range(CW // 16):
                    i = r * (CW // 16) + v
                    sv = ei_v[slot][0, pl.ds(i * 16, 16)]
                    xj = plsc.load_gather(x_v, [sv])
                    av = attr_v[slot][r, pl.ds(v * 16, 16)]
                    msg_v[slot][pl.ds(i * 16, 16)] = xj * av
                    dv = ei_v[slot][1, pl.ds(i * 16, 16)]
                    didx_v[slot][pl.ds(i * 16, 16)] = dv

        def fire_scatters(slot):
            pltpu.async_copy(msg_v[slot],
                             agg_s.at[didx_v[slot]],
                             sc_sem[slot], add=True)

        def drain_scatters(slot):
            pltpu.make_async_copy(msg_v[slot],
                                  agg_s.at[didx_v[slot]],
                                  sc_sem[slot]).wait()

        npair = (BASE_BLK + 2) // 2

        def pair_body(q, carry):
            for slot in (0, 1):
                b = 2 * q + slot

                @pl.when(b < nblk)
                def _():
                    wait_in(slot, blk0 + b)

                    @pl.when(q >= 1)
                    def _():
                        drain_scatters(slot)

                    compute(slot)
                    fire_scatters(slot)

                    @pl.when(b + 2 < nblk)
                    def _():
                        start_in(slot, blk0 + b + 2)

            return carry

        lax.fori_loop(0, npair, pair_body, 0)
        drain_scatters(0)
        drain_scatters(1)

        @pl.when(t == NW - 1)
        def _():
            e0 = NBLK * BLK
            r0 = NBLK * BR
            pltpu.sync_copy(ei_h.at[pl.ds(0, 2), pl.ds(e0, TAIL)],
                            ei_v[0].at[pl.ds(0, 2), pl.ds(0, TAIL)])
            pltpu.sync_copy(attr_h.at[pl.ds(r0, TAIL_R)],
                            attr_v[0].at[pl.ds(0, TAIL_R)])
            for r in range(TAIL_R):
                for v in range(CW // 16):
                    i = r * (CW // 16) + v
                    sv = ei_v[0][0, pl.ds(i * 16, 16)]
                    xj = plsc.load_gather(x_v, [sv])
                    av = attr_v[0][r, pl.ds(v * 16, 16)]
                    msg_v[0][pl.ds(i * 16, 16)] = xj * av
                    dv = ei_v[0][1, pl.ds(i * 16, 16)]
                    didx_v[0][pl.ds(i * 16, 16)] = dv
            pltpu.sync_copy(
                msg_v[0].at[pl.ds(0, TAIL)],
                agg_s.at[didx_v[0].at[pl.ds(0, TAIL)]],
                add=True)

        plsc.subcore_barrier()
        pltpu.sync_copy(agg_s.at[pl.ds(s * ZS, ZS)], z_v)
        pltpu.sync_copy(z_v, out_h.at[pl.ds(c * NP + s * ZS, ZS)])

    return stage1


def _head_groups(nodes_v, par_v, out_v, ngroups):
    lanes38 = lax.iota(jnp.int32, 16) * 38

    def group(j, carry):
        base = j * (16 * 38)
        acc = [par_v[P_B1 + cc] for cc in range(4)]
        for k in range(38):
            nk = plsc.load_gather(nodes_v, [lanes38 + (base + k)])
            for cc in range(4):
                acc[cc] = acc[cc] + par_v[P_W1 + cc * 38 + k] * nk
        h1 = [jnp.maximum(a, 0.0) for a in acc]
        h2 = []
        for cc in range(4):
            a = par_v[P_B2 + cc]
            for dd in range(4):
                a = a + par_v[P_W2 + cc * 4 + dd] * h1[dd]
            h2.append(jnp.maximum(a, 0.0))
        z = par_v[P_B3]
        for cc in range(4):
            z = z + par_v[P_W3 + cc] * h2[cc]
        y = 11.0 / (1.0 + jnp.exp(-z))
        out_v[pl.ds(j * 16, 16)] = y
        return carry

    lax.fori_loop(0, ngroups, group, 0)


def _stage2_tile(part_h, x_h, y_h, a_v, b_v, nodes_v, par_v, out_v,
                 t, n_graphs, n_nodes):
    off = pl.multiple_of(t * NT2, 16)
    g0 = pl.multiple_of(t * GT, 8)
    pltpu.sync_copy(part_h.at[pl.ds(off, n_nodes)],
                    a_v.at[pl.ds(0, n_nodes)])
    pltpu.sync_copy(part_h.at[pl.ds(NP + off, n_nodes)],
                    b_v.at[pl.ds(0, n_nodes)])
    pltpu.sync_copy(x_h.at[pl.ds(off, n_nodes)], nodes_v.at[pl.ds(0, n_nodes)])
    root = par_v[P_ROOT]
    bc = par_v[P_BC]
    zeros16 = jnp.zeros((16,), jnp.float32)

    def comb(i, carry):
        d = pl.ds(i * 16, 16)
        nodes_v[d] = a_v[d] + b_v[d] + nodes_v[d] * root + bc
        return carry

    lax.fori_loop(0, n_nodes // 16, comb, 0)
    for i in range(n_nodes // 16, NBUF // 16):
        nodes_v[pl.ds(i * 16, 16)] = zeros16

    _head_groups(nodes_v, par_v, out_v, (n_graphs + 15) // 16)
    pltpu.sync_copy(out_v.at[pl.ds(0, n_graphs)], y_h.at[pl.ds(g0, n_graphs)])


def _make_stage2():
    mesh = plsc.VectorSubcoreMesh(core_axis_name="c", subcore_axis_name="s")

    @functools.partial(
        pl.kernel,
        out_type=jax.ShapeDtypeStruct((G,), jnp.float32),
        mesh=mesh,
        compiler_params=pltpu.CompilerParams(needs_layout_passes=False),
        scratch_types=[
            pltpu.VMEM((NT2,), jnp.float32),
            pltpu.VMEM((NT2,), jnp.float32),
            pltpu.VMEM((NBUF,), jnp.float32),
            pltpu.VMEM((192, 16), jnp.float32),
            pltpu.VMEM((96,), jnp.float32),
        ],
    )
    def stage2(part_h, x_h, par_h, y_h, a_v, b_v, nodes_v, par_v, out_v):
        c = lax.axis_index("c")
        s = lax.axis_index("s")
        t = c * NS + s
        pltpu.sync_copy(par_h, par_v)

        @pl.when(t < 29)
        def _():
            _stage2_tile(part_h, x_h, y_h, a_v, b_v, nodes_v, par_v, out_v,
                         t, GT, NT2)

        @pl.when(t == 29)
        def _():
            _stage2_tile(part_h, x_h, y_h, a_v, b_v, nodes_v, par_v, out_v,
                         t, G - 29 * GT, (G - 29 * GT) * 38)

    return stage2


_stage1_call = _make_stage1()
_stage2_call = _make_stage2()


def kernel(x, edge_index, edge_attr, batch_vec, We, be, root, bias_conv,
           W1, b1, W2, b2, W3, b3):
    x_f = x.reshape(N)
    ei = edge_index.astype(jnp.int32)
    attr = (edge_attr * We[0, 0] + be[0]).reshape(ATR, CW)
    params = jnp.concatenate([
        We.reshape(-1), be.reshape(-1), root.reshape(-1),
        bias_conv.reshape(-1), W1.reshape(-1), b1, W2.reshape(-1), b2,
        W3.reshape(-1), b3, jnp.zeros((7,), jnp.float32),
    ])
    params = jnp.broadcast_to(params[:, None], (192, 16))
    part = _stage1_call(ei, attr, x_f)
    y = _stage2_call(part, x_f, params)
    return y.reshape(G, 1)


# speedup vs baseline: 1.2273x; 1.2273x over previous
import functools

import jax
import jax.numpy as jnp
from jax import lax
from jax.experimental import pallas as pl
from jax.experimental.pallas import tpu as pltpu
from jax.experimental.pallas import tpu_sc as plsc

N = 100016
E = 3200512
G = 2632

NC = 2
NS = 16
NW = NC * NS

CW = 128
BR = 8
BLK = BR * CW
NBLK = E // BLK
TAIL = E - NBLK * BLK
TAIL_R = TAIL // CW

NB_A = 1562
NB_B = NBLK - NB_A
EA = NB_A * BLK
WROWS_A = EA // CW
WROWS_B = (E - EA) // CW

ZS = 6256
NP = NS * ZS

GT = 88
NT2 = GT * 38
NBUF = 96 * 38

P_WE, P_BE, P_ROOT, P_BC = 0, 1, 2, 3
P_W1, P_B1, P_W2, P_B2, P_W3, P_B3 = 4, 156, 160, 176, 180, 184


def _make_stage1(goff, nb, wrows, has_tail):
    base = nb // NW
    extra = nb - base * NW
    mesh = plsc.VectorSubcoreMesh(core_axis_name="c", subcore_axis_name="s")

    @functools.partial(
        pl.kernel,
        out_type=jax.ShapeDtypeStruct((NC * NP,), jnp.float32),
        mesh=mesh,
        compiler_params=pltpu.CompilerParams(needs_layout_passes=False),
        scratch_types=[
            pltpu.VMEM((N,), jnp.float32),
            [pltpu.VMEM((2, BLK), jnp.int32) for _ in range(2)],
            [pltpu.VMEM((BR, CW), jnp.float32) for _ in range(2)],
            [pltpu.VMEM((BLK,), jnp.float32) for _ in range(2)],
            [pltpu.VMEM((BLK,), jnp.int32) for _ in range(2)],
            pltpu.VMEM((ZS,), jnp.float32),
            pltpu.VMEM_SHARED((NP,), jnp.float32),
            [pltpu.SemaphoreType.DMA for _ in range(2)],
            [pltpu.SemaphoreType.DMA for _ in range(2)],
        ],
    )
    def stage1(ei_h, w_h, x_h, out_h,
               x_v, ei_v, w_v, msg_v, didx_v, z_v, agg_s, in_sem, sc_sem):
        c = lax.axis_index("c")
        s = lax.axis_index("s")
        t = c * NS + s

        nblk = jnp.where(t < extra, base + 1, base)
        blk0 = t * base + jnp.minimum(t, extra)

        def start_in(slot, lblk):
            e0 = pl.multiple_of((goff + lblk) * BLK, BLK)
            r0 = pl.multiple_of(lblk * BR, BR)
            pltpu.async_copy(ei_h.at[pl.ds(0, 2), pl.ds(e0, BLK)],
                             ei_v[slot], in_sem[slot])
            pltpu.async_copy(w_h.at[pl.ds(r0, BR)],
                             w_v[slot], in_sem[slot])

        def wait_in(slot, lblk):
            e0 = pl.multiple_of((goff + lblk) * BLK, BLK)
            r0 = pl.multiple_of(lblk * BR, BR)
            pltpu.make_async_copy(ei_h.at[pl.ds(0, 2), pl.ds(e0, BLK)],
                                  ei_v[slot], in_sem[slot]).wait()
            pltpu.make_async_copy(w_h.at[pl.ds(r0, BR)],
                                  w_v[slot], in_sem[slot]).wait()

        start_in(0, blk0)
        start_in(1, blk0 + 1)

        zeros16 = jnp.zeros((16,), jnp.float32)

        def zbody(i, carry):
            z_v[pl.ds(i * 16, 16)] = zeros16
            return carry

        lax.fori_loop(0, ZS // 16, zbody, 0)
        pltpu.sync_copy(z_v, agg_s.at[pl.ds(s * ZS, ZS)])
        plsc.subcore_barrier()

        pltpu.sync_copy(x_h, x_v)

        def compute(slot):
            for r in range(BR):
                for v in range(CW // 16):
                    i = r * (CW // 16) + v
                    sv = ei_v[slot][0, pl.ds(i * 16, 16)]
                    xj = plsc.load_gather(x_v, [sv])
                    wv = w_v[slot][r, pl.ds(v * 16, 16)]
                    msg_v[slot][pl.ds(i * 16, 16)] = xj * wv
                    dv = ei_v[slot][1, pl.ds(i * 16, 16)]
                    didx_v[slot][pl.ds(i * 16, 16)] = dv

        def fire_scatters(slot):
            pltpu.async_copy(msg_v[slot], agg_s.at[didx_v[slot]],
                             sc_sem[slot], add=True)

        def drain_scatters(slot):
            pltpu.make_async_copy(msg_v[slot], agg_s.at[didx_v[slot]],
                                  sc_sem[slot]).wait()

        npair = (base + 2) // 2

        def pair_body(q, carry):
            for slot in (0, 1):
                b = 2 * q + slot

                @pl.when(b < nblk)
                def _():
                    wait_in(slot, blk0 + b)

                    @pl.when(q >= 1)
                    def _():
                        drain_scatters(slot)

                    compute(slot)
                    fire_scatters(slot)

                    @pl.when(b + 2 < nblk)
                    def _():
                        start_in(slot, blk0 + b + 2)

            return carry

        lax.fori_loop(0, npair, pair_body, 0)
        drain_scatters(0)
        drain_scatters(1)

        if has_tail:
            @pl.when(t == NW - 1)
            def _():
                e0 = NBLK * BLK
                r0 = nb * BR
                pltpu.sync_copy(ei_h.at[pl.ds(0, 2), pl.ds(e0, TAIL)],
                                ei_v[0].at[pl.ds(0, 2), pl.ds(0, TAIL)])
                pltpu.sync_copy(w_h.at[pl.ds(r0, TAIL_R)],
                                w_v[0].at[pl.ds(0, TAIL_R)])
                for r in range(TAIL_R):
                    for v in range(CW // 16):
                        i = r * (CW // 16) + v
                        sv = ei_v[0][0, pl.ds(i * 16, 16)]
                        xj = plsc.load_gather(x_v, [sv])
                        wv = w_v[0][r, pl.ds(v * 16, 16)]
                        msg_v[0][pl.ds(i * 16, 16)] = xj * wv
                        dv = ei_v[0][1, pl.ds(i * 16, 16)]
                        didx_v[0][pl.ds(i * 16, 16)] = dv
                pltpu.sync_copy(msg_v[0].at[pl.ds(0, TAIL)],
                                agg_s.at[didx_v[0].at[pl.ds(0, TAIL)]],
                                add=True)

        plsc.subcore_barrier()
        pltpu.sync_copy(agg_s.at[pl.ds(s * ZS, ZS)], z_v)
        pltpu.sync_copy(z_v, out_h.at[pl.ds(c * NP + s * ZS, ZS)])

    return stage1


def _head_groups(nodes_v, par_v, out_v, ngroups):
    lanes38 = lax.iota(jnp.int32, 16) * 38

    def group(j, carry):
        base = j * (16 * 38)
        acc = [par_v[P_B1 + cc] for cc in range(4)]
        for k in range(38):
            nk = plsc.load_gather(nodes_v, [lanes38 + (base + k)])
            for cc in range(4):
                acc[cc] = acc[cc] + par_v[P_W1 + cc * 38 + k] * nk
        h1 = [jnp.maximum(a, 0.0) for a in acc]
        h2 = []
        for cc in range(4):
            a = par_v[P_B2 + cc]
            for dd in range(4):
                a = a + par_v[P_W2 + cc * 4 + dd] * h1[dd]
            h2.append(jnp.maximum(a, 0.0))
        z = par_v[P_B3]
        for cc in range(4):
            z = z + par_v[P_W3 + cc] * h2[cc]
        y = 11.0 / (1.0 + jnp.exp(-z))
        out_v[pl.ds(j * 16, 16)] = y
        return carry

    lax.fori_loop(0, ngroups, group, 0)


def _stage2_tile(pa_h, pb_h, x_h, y_h, a_v, b_v, c_v, d_v, nodes_v, par_v,
                 out_v, t, n_graphs, n_nodes):
    off = pl.multiple_of(t * NT2, 16)
    g0 = pl.multiple_of(t * GT, 8)
    pltpu.sync_copy(pa_h.at[pl.ds(off, n_nodes)], a_v.at[pl.ds(0, n_nodes)])
    pltpu.sync_copy(pa_h.at[pl.ds(NP + off, n_nodes)],
                    b_v.at[pl.ds(0, n_nodes)])
    pltpu.sync_copy(pb_h.at[pl.ds(off, n_nodes)], c_v.at[pl.ds(0, n_nodes)])
    pltpu.sync_copy(pb_h.at[pl.ds(NP + off, n_nodes)],
                    d_v.at[pl.ds(0, n_nodes)])
    pltpu.sync_copy(x_h.at[pl.ds(off, n_nodes)], nodes_v.at[pl.ds(0, n_nodes)])
    root = par_v[P_ROOT]
    bc = par_v[P_BC]
    zeros16 = jnp.zeros((16,), jnp.float32)

    def comb(i, carry):
        d = pl.ds(i * 16, 16)
        nodes_v[d] = (a_v[d] + b_v[d]) + (c_v[d] + d_v[d]) \
            + nodes_v[d] * root + bc
        return carry

    lax.fori_loop(0, n_nodes // 16, comb, 0)
    for i in range(n_nodes // 16, NBUF // 16):
        nodes_v[pl.ds(i * 16, 16)] = zeros16

    _head_groups(nodes_v, par_v, out_v, (n_graphs + 15) // 16)
    pltpu.sync_copy(out_v.at[pl.ds(0, n_graphs)], y_h.at[pl.ds(g0, n_graphs)])


def _make_stage2():
    mesh = plsc.VectorSubcoreMesh(core_axis_name="c", subcore_axis_name="s")

    @functools.partial(
        pl.kernel,
        out_type=jax.ShapeDtypeStruct((G,), jnp.float32),
        mesh=mesh,
        compiler_params=pltpu.CompilerParams(needs_layout_passes=False),
        scratch_types=[
            pltpu.VMEM((NT2,), jnp.float32),
            pltpu.VMEM((NT2,), jnp.float32),
            pltpu.VMEM((NT2,), jnp.float32),
            pltpu.VMEM((NT2,), jnp.float32),
            pltpu.VMEM((NBUF,), jnp.float32),
            pltpu.VMEM((192, 16), jnp.float32),
            pltpu.VMEM((96,), jnp.float32),
        ],
    )
    def stage2(pa_h, pb_h, x_h, par_h, y_h,
               a_v, b_v, c_v, d_v, nodes_v, par_v, out_v):
        c = lax.axis_index("c")
        s = lax.axis_index("s")
        t = c * NS + s
        pltpu.sync_copy(par_h, par_v)

        @pl.when(t < 29)
        def _():
            _stage2_tile(pa_h, pb_h, x_h, y_h, a_v, b_v, c_v, d_v, nodes_v,
                         par_v, out_v, t, GT, NT2)

        @pl.when(t == 29)
        def _():
            _stage2_tile(pa_h, pb_h, x_h, y_h, a_v, b_v, c_v, d_v, nodes_v,
                         par_v, out_v, t, G - 29 * GT, (G - 29 * GT) * 38)

    return stage2


_stage1a_call = _make_stage1(0, NB_A, WROWS_A, False)
_stage1b_call = _make_stage1(NB_A, NB_B, WROWS_B, True)
_stage2_call = _make_stage2()


def kernel(x, edge_index, edge_attr, batch_vec, We, be, root, bias_conv,
           W1, b1, W2, b2, W3, b3):
    x_f = x.reshape(N)
    ei = edge_index.astype(jnp.int32)
    w_a = (edge_attr[:EA] * We[0, 0] + be[0]).reshape(WROWS_A, CW)
    w_b = (edge_attr[EA:] * We[0, 0] + be[0]).reshape(WROWS_B, CW)
    params = jnp.concatenate([
        We.reshape(-1), be.reshape(-1), root.reshape(-1),
        bias_conv.reshape(-1), W1.reshape(-1), b1, W2.reshape(-1), b2,
        W3.reshape(-1), b3, jnp.zeros((7,), jnp.float32),
    ])
    params = jnp.broadcast_to(params[:, None], (192, 16))
    part_a = _stage1a_call(ei, w_a, x_f)
    part_b = _stage1b_call(ei, w_b, x_f)
    y = _stage2_call(part_a, part_b, x_f, params)
    return y.reshape(G, 1)


# speedup vs baseline: 1.6496x; 1.3441x over previous
import functools

import jax
import jax.numpy as jnp
from jax import lax
from jax.experimental import pallas as pl
from jax.experimental.pallas import tpu as pltpu
from jax.experimental.pallas import tpu_sc as plsc

N = 100016
E = 3200512
G = 2632

NC = 2
NS = 16
NW = NC * NS

CW = 128
BR = 8
BLK = BR * CW
NBLK = E // BLK
TAIL = E - NBLK * BLK
TAIL_R = TAIL // CW

NB_A = 1894
NB_B = NBLK - NB_A
EA = NB_A * BLK
WROWS_A = EA // CW
WROWS_B = (E - EA) // CW

ZS = 6256
NP = NS * ZS

GT = 88
NT2 = GT * 38
NBUF = 96 * 38

P_WE, P_BE, P_ROOT, P_BC = 0, 1, 2, 3
P_W1, P_B1, P_W2, P_B2, P_W3, P_B3 = 4, 156, 160, 176, 180, 184


def _make_stage1(goff, nb, wrows, has_tail):
    base = nb // NW
    extra = nb - base * NW
    mesh = plsc.VectorSubcoreMesh(core_axis_name="c", subcore_axis_name="s")

    @functools.partial(
        pl.kernel,
        out_type=jax.ShapeDtypeStruct((NC * NP,), jnp.float32),
        mesh=mesh,
        compiler_params=pltpu.CompilerParams(needs_layout_passes=False),
        scratch_types=[
            pltpu.VMEM((N,), jnp.float32),
            [pltpu.VMEM((2, BLK), jnp.int32) for _ in range(2)],
            [pltpu.VMEM((BR, CW), jnp.float32) for _ in range(2)],
            [pltpu.VMEM((BLK,), jnp.float32) for _ in range(2)],
            [pltpu.VMEM((BLK,), jnp.int32) for _ in range(2)],
            pltpu.VMEM((ZS,), jnp.float32),
            pltpu.VMEM_SHARED((NP,), jnp.float32),
            [pltpu.SemaphoreType.DMA for _ in range(2)],
            [pltpu.SemaphoreType.DMA for _ in range(2)],
        ],
    )
    def stage1(ei_h, w_h, x_h, out_h,
               x_v, ei_v, w_v, msg_v, didx_v, z_v, agg_s, in_sem, sc_sem):
        c = lax.axis_index("c")
        s = lax.axis_index("s")
        t = c * NS + s

        nblk = jnp.where(t < extra, base + 1, base)
        blk0 = t * base + jnp.minimum(t, extra)

        def start_in(slot, lblk):
            e0 = pl.multiple_of((goff + lblk) * BLK, BLK)
            r0 = pl.multiple_of(lblk * BR, BR)
            pltpu.async_copy(ei_h.at[pl.ds(0, 2), pl.ds(e0, BLK)],
                             ei_v[slot], in_sem[slot])
            pltpu.async_copy(w_h.at[pl.ds(r0, BR)],
                             w_v[slot], in_sem[slot])

        def wait_in(slot, lblk):
            e0 = pl.multiple_of((goff + lblk) * BLK, BLK)
            r0 = pl.multiple_of(lblk * BR, BR)
            pltpu.make_async_copy(ei_h.at[pl.ds(0, 2), pl.ds(e0, BLK)],
                                  ei_v[slot], in_sem[slot]).wait()
            pltpu.make_async_copy(w_h.at[pl.ds(r0, BR)],
                                  w_v[slot], in_sem[slot]).wait()

        start_in(0, blk0)
        start_in(1, blk0 + 1)

        zeros16 = jnp.zeros((16,), jnp.float32)

        def zbody(i, carry):
            z_v[pl.ds(i * 16, 16)] = zeros16
            return carry

        lax.fori_loop(0, ZS // 16, zbody, 0)
        pltpu.sync_copy(z_v, agg_s.at[pl.ds(s * ZS, ZS)])
        plsc.subcore_barrier()

        pltpu.sync_copy(x_h, x_v)

        def compute(slot):
            for r in range(BR):
                for v in range(CW // 16):
                    i = r * (CW // 16) + v
                    sv = ei_v[slot][0, pl.ds(i * 16, 16)]
                    xj = plsc.load_gather(x_v, [sv])
                    wv = w_v[slot][r, pl.ds(v * 16, 16)]
                    msg_v[slot][pl.ds(i * 16, 16)] = xj * wv
                    dv = ei_v[slot][1, pl.ds(i * 16, 16)]
                    didx_v[slot][pl.ds(i * 16, 16)] = dv

        def fire_scatters(slot):
            pltpu.async_copy(msg_v[slot], agg_s.at[didx_v[slot]],
                             sc_sem[slot], add=True)

        def drain_scatters(slot):
            pltpu.make_async_copy(msg_v[slot], agg_s.at[didx_v[slot]],
                                  sc_sem[slot]).wait()

        npair = (base + 2) // 2

        def pair_body(q, carry):
            for slot in (0, 1):
                b = 2 * q + slot

                @pl.when(b < nblk)
                def _():
                    wait_in(slot, blk0 + b)

                    @pl.when(q >= 1)
                    def _():
                        drain_scatters(slot)

                    compute(slot)
                    fire_scatters(slot)

                    @pl.when(b + 2 < nblk)
                    def _():
                        start_in(slot, blk0 + b + 2)

            return carry

        lax.fori_loop(0, npair, pair_body, 0)
        drain_scatters(0)
        drain_scatters(1)

        if has_tail:
            @pl.when(t == NW - 1)
            def _():
                e0 = NBLK * BLK
                r0 = nb * BR
                pltpu.sync_copy(ei_h.at[pl.ds(0, 2), pl.ds(e0, TAIL)],
                                ei_v[0].at[pl.ds(0, 2), pl.ds(0, TAIL)])
                pltpu.sync_copy(w_h.at[pl.ds(r0, TAIL_R)],
                                w_v[0].at[pl.ds(0, TAIL_R)])
                for r in range(TAIL_R):
                    for v in range(CW // 16):
                        i = r * (CW // 16) + v
                        sv = ei_v[0][0, pl.ds(i * 16, 16)]
                        xj = plsc.load_gather(x_v, [sv])
                        wv = w_v[0][r, pl.ds(v * 16, 16)]
                        msg_v[0][pl.ds(i * 16, 16)] = xj * wv
                        dv = ei_v[0][1, pl.ds(i * 16, 16)]
                        didx_v[0][pl.ds(i * 16, 16)] = dv
                pltpu.sync_copy(msg_v[0].at[pl.ds(0, TAIL)],
                                agg_s.at[didx_v[0].at[pl.ds(0, TAIL)]],
                                add=True)

        plsc.subcore_barrier()
        pltpu.sync_copy(agg_s.at[pl.ds(s * ZS, ZS)], z_v)
        pltpu.sync_copy(z_v, out_h.at[pl.ds(c * NP + s * ZS, ZS)])

    return stage1


def _head_groups(nodes_v, par_v, out_v, ngroups):
    lanes38 = lax.iota(jnp.int32, 16) * 38

    def group(j, carry):
        base = j * (16 * 38)
        acc = [par_v[P_B1 + cc] for cc in range(4)]
        for k in range(38):
            nk = plsc.load_gather(nodes_v, [lanes38 + (base + k)])
            for cc in range(4):
                acc[cc] = acc[cc] + par_v[P_W1 + cc * 38 + k] * nk
        h1 = [jnp.maximum(a, 0.0) for a in acc]
        h2 = []
        for cc in range(4):
            a = par_v[P_B2 + cc]
            for dd in range(4):
                a = a + par_v[P_W2 + cc * 4 + dd] * h1[dd]
            h2.append(jnp.maximum(a, 0.0))
        z = par_v[P_B3]
        for cc in range(4):
            z = z + par_v[P_W3 + cc] * h2[cc]
        y = 11.0 / (1.0 + jnp.exp(-z))
        out_v[pl.ds(j * 16, 16)] = y
        return carry

    lax.fori_loop(0, ngroups, group, 0)


def _stage2_tile(pa_h, pb_h, x_h, y_h, a_v, b_v, c_v, d_v, nodes_v, par_v,
                 out_v, t, n_graphs, n_nodes):
    off = pl.multiple_of(t * NT2, 16)
    g0 = pl.multiple_of(t * GT, 8)
    pltpu.sync_copy(pa_h.at[pl.ds(off, n_nodes)], a_v.at[pl.ds(0, n_nodes)])
    pltpu.sync_copy(pa_h.at[pl.ds(NP + off, n_nodes)],
                    b_v.at[pl.ds(0, n_nodes)])
    pltpu.sync_copy(pb_h.at[pl.ds(off, n_nodes)], c_v.at[pl.ds(0, n_nodes)])
    pltpu.sync_copy(pb_h.at[pl.ds(NP + off, n_nodes)],
                    d_v.at[pl.ds(0, n_nodes)])
    pltpu.sync_copy(x_h.at[pl.ds(off, n_nodes)], nodes_v.at[pl.ds(0, n_nodes)])
    root = par_v[P_ROOT]
    bc = par_v[P_BC]
    zeros16 = jnp.zeros((16,), jnp.float32)

    def comb(i, carry):
        d = pl.ds(i * 16, 16)
        nodes_v[d] = (a_v[d] + b_v[d]) + (c_v[d] + d_v[d]) \
            + nodes_v[d] * root + bc
        return carry

    lax.fori_loop(0, n_nodes // 16, comb, 0)
    for i in range(n_nodes // 16, NBUF // 16):
        nodes_v[pl.ds(i * 16, 16)] = zeros16

    _head_groups(nodes_v, par_v, out_v, (n_graphs + 15) // 16)
    pltpu.sync_copy(out_v.at[pl.ds(0, n_graphs)], y_h.at[pl.ds(g0, n_graphs)])


def _make_stage2():
    mesh = plsc.VectorSubcoreMesh(core_axis_name="c", subcore_axis_name="s")

    @functools.partial(
        pl.kernel,
        out_type=jax.ShapeDtypeStruct((G,), jnp.float32),
        mesh=mesh,
        compiler_params=pltpu.CompilerParams(needs_layout_passes=False),
        scratch_types=[
            pltpu.VMEM((NT2,), jnp.float32),
            pltpu.VMEM((NT2,), jnp.float32),
            pltpu.VMEM((NT2,), jnp.float32),
            pltpu.VMEM((NT2,), jnp.float32),
            pltpu.VMEM((NBUF,), jnp.float32),
            pltpu.VMEM((192, 16), jnp.float32),
            pltpu.VMEM((96,), jnp.float32),
        ],
    )
    def stage2(pa_h, pb_h, x_h, par_h, y_h,
               a_v, b_v, c_v, d_v, nodes_v, par_v, out_v):
        c = lax.axis_index("c")
        s = lax.axis_index("s")
        t = c * NS + s
        pltpu.sync_copy(par_h, par_v)

        @pl.when(t < 29)
        def _():
            _stage2_tile(pa_h, pb_h, x_h, y_h, a_v, b_v, c_v, d_v, nodes_v,
                         par_v, out_v, t, GT, NT2)

        @pl.when(t == 29)
        def _():
            _stage2_tile(pa_h, pb_h, x_h, y_h, a_v, b_v, c_v, d_v, nodes_v,
                         par_v, out_v, t, G - 29 * GT, (G - 29 * GT) * 38)

    return stage2


_stage1a_call = _make_stage1(0, NB_A, WROWS_A, False)
_stage1b_call = _make_stage1(NB_A, NB_B, WROWS_B, True)
_stage2_call = _make_stage2()


def kernel(x, edge_index, edge_attr, batch_vec, We, be, root, bias_conv,
           W1, b1, W2, b2, W3, b3):
    x_f = x.reshape(N)
    ei = edge_index.astype(jnp.int32)
    w = edge_attr * We[0, 0] + be[0]
    w_a = w[:EA].reshape(WROWS_A, CW)
    w_b = w[EA:].reshape(WROWS_B, CW)
    params = jnp.concatenate([
        We.reshape(-1), be.reshape(-1), root.reshape(-1),
        bias_conv.reshape(-1), W1.reshape(-1), b1, W2.reshape(-1), b2,
        W3.reshape(-1), b3, jnp.zeros((7,), jnp.float32),
    ])
    params = jnp.broadcast_to(params[:, None], (192, 16))
    part_a = _stage1a_call(ei, w_a, x_f)
    part_b = _stage1b_call(ei, w_b, x_f)
    y = _stage2_call(part_a, part_b, x_f, params)
    return y.reshape(G, 1)


# speedup vs baseline: 1.6915x; 1.0254x over previous
import functools

import jax
import jax.numpy as jnp
from jax import lax
from jax.experimental import pallas as pl
from jax.experimental.pallas import tpu as pltpu
from jax.experimental.pallas import tpu_sc as plsc

N = 100016
E = 3200512
G = 2632

NC = 2
NS = 16
NW = NC * NS

CW = 128
BR = 8
BLK = BR * CW
NBLK = E // BLK
TAIL = E - NBLK * BLK
TAIL_R = TAIL // CW

NB_A = 1969
NB_B = NBLK - NB_A
EA = NB_A * BLK
WROWS_A = EA // CW
WROWS_B = (E - EA) // CW

ZS = 6256
NP = NS * ZS

GT = 88
NT2 = GT * 38
NBUF = 96 * 38

P_WE, P_BE, P_ROOT, P_BC = 0, 1, 2, 3
P_W1, P_B1, P_W2, P_B2, P_W3, P_B3 = 4, 156, 160, 176, 180, 184


def _make_stage1(goff, nb, wrows, has_tail):
    base = nb // NW
    extra = nb - base * NW
    mesh = plsc.VectorSubcoreMesh(core_axis_name="c", subcore_axis_name="s")

    @functools.partial(
        pl.kernel,
        out_type=jax.ShapeDtypeStruct((NC * NP,), jnp.float32),
        mesh=mesh,
        compiler_params=pltpu.CompilerParams(needs_layout_passes=False),
        scratch_types=[
            pltpu.VMEM((N,), jnp.float32),
            [pltpu.VMEM((2, BLK), jnp.int32) for _ in range(2)],
            [pltpu.VMEM((BR, CW), jnp.float32) for _ in range(2)],
            [pltpu.VMEM((BLK,), jnp.float32) for _ in range(2)],
            [pltpu.VMEM((BLK,), jnp.int32) for _ in range(2)],
            pltpu.VMEM((ZS,), jnp.float32),
            pltpu.VMEM_SHARED((NP,), jnp.float32),
            [pltpu.SemaphoreType.DMA for _ in range(2)],
            [pltpu.SemaphoreType.DMA for _ in range(2)],
        ],
    )
    def stage1(ei_h, w_h, x_h, out_h,
               x_v, ei_v, w_v, msg_v, didx_v, z_v, agg_s, in_sem, sc_sem):
        c = lax.axis_index("c")
        s = lax.axis_index("s")
        t = c * NS + s

        nblk = jnp.where(t < extra, base + 1, base)
        blk0 = t * base + jnp.minimum(t, extra)

        def start_in(slot, lblk):
            e0 = pl.multiple_of((goff + lblk) * BLK, BLK)
            r0 = pl.multiple_of(lblk * BR, BR)
            pltpu.async_copy(ei_h.at[pl.ds(0, 2), pl.ds(e0, BLK)],
                             ei_v[slot], in_sem[slot])
            pltpu.async_copy(w_h.at[pl.ds(r0, BR)],
                             w_v[slot], in_sem[slot])

        def wait_in(slot, lblk):
            e0 = pl.multiple_of((goff + lblk) * BLK, BLK)
            r0 = pl.multiple_of(lblk * BR, BR)
            pltpu.make_async_copy(ei_h.at[pl.ds(0, 2), pl.ds(e0, BLK)],
                                  ei_v[slot], in_sem[slot]).wait()
            pltpu.make_async_copy(w_h.at[pl.ds(r0, BR)],
                                  w_v[slot], in_sem[slot]).wait()

        start_in(0, blk0)
        start_in(1, blk0 + 1)

        zeros16 = jnp.zeros((16,), jnp.float32)

        def zbody(i, carry):
            z_v[pl.ds(i * 16, 16)] = zeros16
            return carry

        lax.fori_loop(0, ZS // 16, zbody, 0)
        pltpu.sync_copy(z_v, agg_s.at[pl.ds(s * ZS, ZS)])
        plsc.subcore_barrier()

        pltpu.sync_copy(x_h, x_v)

        def compute(slot):
            for r in range(BR):
                for v in range(CW // 16):
                    i = r * (CW // 16) + v
                    sv = ei_v[slot][0, pl.ds(i * 16, 16)]
                    xj = plsc.load_gather(x_v, [sv])
                    wv = w_v[slot][r, pl.ds(v * 16, 16)]
                    msg_v[slot][pl.ds(i * 16, 16)] = xj * wv
                    dv = ei_v[slot][1, pl.ds(i * 16, 16)]
                    didx_v[slot][pl.ds(i * 16, 16)] = dv

        def fire_scatters(slot):
            pltpu.async_copy(msg_v[slot], agg_s.at[didx_v[slot]],
                             sc_sem[slot], add=True)

        def drain_scatters(slot):
            pltpu.make_async_copy(msg_v[slot], agg_s.at[didx_v[slot]],
                                  sc_sem[slot]).wait()

        npair = (base + 2) // 2

        def pair_body(q, carry):
            for slot in (0, 1):
                b = 2 * q + slot

                @pl.when(b < nblk)
                def _():
                    wait_in(slot, blk0 + b)

                    @pl.when(q >= 1)
                    def _():
                        drain_scatters(slot)

                    compute(slot)
                    fire_scatters(slot)

                    @pl.when(b + 2 < nblk)
                    def _():
                        start_in(slot, blk0 + b + 2)

            return carry

        lax.fori_loop(0, npair, pair_body, 0)
        drain_scatters(0)
        drain_scatters(1)

        if has_tail:
            @pl.when(t == NW - 1)
            def _():
                e0 = NBLK * BLK
                r0 = nb * BR
                pltpu.sync_copy(ei_h.at[pl.ds(0, 2), pl.ds(e0, TAIL)],
                                ei_v[0].at[pl.ds(0, 2), pl.ds(0, TAIL)])
                pltpu.sync_copy(w_h.at[pl.ds(r0, TAIL_R)],
                                w_v[0].at[pl.ds(0, TAIL_R)])
                for r in range(TAIL_R):
                    for v in range(CW // 16):
                        i = r * (CW // 16) + v
                        sv = ei_v[0][0, pl.ds(i * 16, 16)]
                        xj = plsc.load_gather(x_v, [sv])
                        wv = w_v[0][r, pl.ds(v * 16, 16)]
                        msg_v[0][pl.ds(i * 16, 16)] = xj * wv
                        dv = ei_v[0][1, pl.ds(i * 16, 16)]
                        didx_v[0][pl.ds(i * 16, 16)] = dv
                pltpu.sync_copy(msg_v[0].at[pl.ds(0, TAIL)],
                                agg_s.at[didx_v[0].at[pl.ds(0, TAIL)]],
                                add=True)

        plsc.subcore_barrier()
        pltpu.sync_copy(agg_s.at[pl.ds(s * ZS, ZS)], z_v)
        pltpu.sync_copy(z_v, out_h.at[pl.ds(c * NP + s * ZS, ZS)])

    return stage1


def _head_groups(nodes_v, par_v, out_v, ngroups):
    lanes38 = lax.iota(jnp.int32, 16) * 38

    def group(j, carry):
        base = j * (16 * 38)
        acc = [par_v[P_B1 + cc] for cc in range(4)]
        for k in range(38):
            nk = plsc.load_gather(nodes_v, [lanes38 + (base + k)])
            for cc in range(4):
                acc[cc] = acc[cc] + par_v[P_W1 + cc * 38 + k] * nk
        h1 = [jnp.maximum(a, 0.0) for a in acc]
        h2 = []
        for cc in range(4):
            a = par_v[P_B2 + cc]
            for dd in range(4):
                a = a + par_v[P_W2 + cc * 4 + dd] * h1[dd]
            h2.append(jnp.maximum(a, 0.0))
        z = par_v[P_B3]
        for cc in range(4):
            z = z + par_v[P_W3 + cc] * h2[cc]
        y = 11.0 / (1.0 + jnp.exp(-z))
        out_v[pl.ds(j * 16, 16)] = y
        return carry

    lax.fori_loop(0, ngroups, group, 0)


def _stage2_tile(pa_h, pb_h, x_h, y_h, a_v, b_v, c_v, d_v, nodes_v, par_v,
                 out_v, ld_sem, t, n_graphs, n_nodes):
    off = pl.multiple_of(t * NT2, 16)
    g0 = pl.multiple_of(t * GT, 8)
    loads = [
        (pa_h.at[pl.ds(off, n_nodes)], a_v.at[pl.ds(0, n_nodes)]),
        (pa_h.at[pl.ds(NP + off, n_nodes)], b_v.at[pl.ds(0, n_nodes)]),
        (pb_h.at[pl.ds(off, n_nodes)], c_v.at[pl.ds(0, n_nodes)]),
        (pb_h.at[pl.ds(NP + off, n_nodes)], d_v.at[pl.ds(0, n_nodes)]),
        (x_h.at[pl.ds(off, n_nodes)], nodes_v.at[pl.ds(0, n_nodes)]),
    ]
    for src, dst in loads:
        pltpu.async_copy(src, dst, ld_sem)
    for src, dst in loads:
        pltpu.make_async_copy(src, dst, ld_sem).wait()
    root = par_v[P_ROOT]
    bc = par_v[P_BC]
    zeros16 = jnp.zeros((16,), jnp.float32)

    def comb(i, carry):
        d = pl.ds(i * 16, 16)
        nodes_v[d] = (a_v[d] + b_v[d]) + (c_v[d] + d_v[d]) \
            + nodes_v[d] * root + bc
        return carry

    lax.fori_loop(0, n_nodes // 16, comb, 0)
    for i in range(n_nodes // 16, NBUF // 16):
        nodes_v[pl.ds(i * 16, 16)] = zeros16

    _head_groups(nodes_v, par_v, out_v, (n_graphs + 15) // 16)
    pltpu.sync_copy(out_v.at[pl.ds(0, n_graphs)], y_h.at[pl.ds(g0, n_graphs)])


def _make_stage2():
    mesh = plsc.VectorSubcoreMesh(core_axis_name="c", subcore_axis_name="s")

    @functools.partial(
        pl.kernel,
        out_type=jax.ShapeDtypeStruct((G,), jnp.float32),
        mesh=mesh,
        compiler_params=pltpu.CompilerParams(needs_layout_passes=False),
        scratch_types=[
            pltpu.VMEM((NT2,), jnp.float32),
            pltpu.VMEM((NT2,), jnp.float32),
            pltpu.VMEM((NT2,), jnp.float32),
            pltpu.VMEM((NT2,), jnp.float32),
            pltpu.VMEM((NBUF,), jnp.float32),
            pltpu.VMEM((192, 16), jnp.float32),
            pltpu.VMEM((96,), jnp.float32),
            pltpu.SemaphoreType.DMA,
        ],
    )
    def stage2(pa_h, pb_h, x_h, par_h, y_h,
               a_v, b_v, c_v, d_v, nodes_v, par_v, out_v, ld_sem):
        c = lax.axis_index("c")
        s = lax.axis_index("s")
        t = c * NS + s
        pltpu.sync_copy(par_h, par_v)

        @pl.when(t < 29)
        def _():
            _stage2_tile(pa_h, pb_h, x_h, y_h, a_v, b_v, c_v, d_v, nodes_v,
                         par_v, out_v, ld_sem, t, GT, NT2)

        @pl.when(t == 29)
        def _():
            _stage2_tile(pa_h, pb_h, x_h, y_h, a_v, b_v, c_v, d_v, nodes_v,
                         par_v, out_v, ld_sem, t, G - 29 * GT,
                         (G - 29 * GT) * 38)

    return stage2


_stage1a_call = _make_stage1(0, NB_A, WROWS_A, False)
_stage1b_call = _make_stage1(NB_A, NB_B, WROWS_B, True)
_stage2_call = _make_stage2()


def kernel(x, edge_index, edge_attr, batch_vec, We, be, root, bias_conv,
           W1, b1, W2, b2, W3, b3):
    x_f = x.reshape(N)
    ei = edge_index.astype(jnp.int32)
    w = edge_attr * We[0, 0] + be[0]
    w_a = w[:EA].reshape(WROWS_A, CW)
    w_b = w[EA:].reshape(WROWS_B, CW)
    params = jnp.concatenate([
        We.reshape(-1), be.reshape(-1), root.reshape(-1),
        bias_conv.reshape(-1), W1.reshape(-1), b1, W2.reshape(-1), b2,
        W3.reshape(-1), b3, jnp.zeros((7,), jnp.float32),
    ])
    params = jnp.broadcast_to(params[:, None], (192, 16))
    part_a = _stage1a_call(ei, w_a, x_f)
    part_b = _stage1b_call(ei, w_b, x_f)
    y = _stage2_call(part_a, part_b, x_f, params)
    return y.reshape(G, 1)
